# Initial kernel scaffold; baseline (speedup 1.0000x reference)
import jax
import jax.numpy as jnp
from jax.experimental import pallas as pl


def kernel(x, params, batch):
    raise NotImplementedError("write your pallas kernel here")


# speedup vs baseline: 7.1721x; 7.1721x over previous
import functools

import jax
import jax.numpy as jnp
from jax import lax
from jax.experimental import pallas as pl
from jax.experimental.pallas import tpu as pltpu
from jax.experimental.pallas import tpu_sc as plsc

N = 8192
NB = 8
K = 6
SPACE_D = 16
PROP_D = 64
GRAV_DIM = 64
EPS = 1e-5
T = 128

_PREC = lax.Precision.DEFAULT
_HI = lax.Precision.HIGHEST


def _dot_t(a, b, precision=_PREC):
    return lax.dot_general(a, b, (((1,), (1,)), ((), ())),
                           precision=precision,
                           preferred_element_type=jnp.float32)


def _dot(a, b, precision=_PREC):
    return lax.dot_general(a, b, (((1,), (0,)), ((), ())),
                           precision=precision,
                           preferred_element_type=jnp.float32)


def _bn(y, w, b):
    m = jnp.mean(y, axis=0, keepdims=True)
    v = jnp.mean((y - m) ** 2, axis=0, keepdims=True)
    return (y - m) / jnp.sqrt(v + EPS) * w + b


def _exchange_body(x_ref, bcol_ref, brow_ref, o_ref):
    x = x_ref[...]
    bcol = bcol_ref[...]
    brow = brow_ref[...]
    e_row = lax.broadcasted_iota(jnp.int32, (NB, N), 0)
    oh_t = (e_row == brow).astype(jnp.float32)
    sums = _dot(oh_t, x, precision=_HI)
    cnt = jnp.sum(oh_t, axis=1, keepdims=True)
    mean = sums / jnp.maximum(cnt, 1.0)
    mns, mxs = [], []
    for e in range(NB):
        mask = bcol == e
        mns.append(jnp.min(jnp.where(mask, x, jnp.inf), axis=0, keepdims=True))
        mxs.append(jnp.max(jnp.where(mask, x, -jnp.inf), axis=0, keepdims=True))
    mmm = jnp.concatenate([mean] + [jnp.concatenate(mns, axis=0),
                                    jnp.concatenate(mxs, axis=0)], axis=1)
    oh = (bcol == lax.broadcasted_iota(jnp.int32, (N, NB), 1)).astype(jnp.float32)
    o_ref[...] = jnp.concatenate([_dot(oh, mmm, precision=_HI), x], axis=1)


def _exchange(x, bcol, brow):
    c = x.shape[1]
    return pl.pallas_call(
        _exchange_body,
        out_shape=jax.ShapeDtypeStruct((N, 4 * c), jnp.float32),
    )(x, bcol, brow)


def _pre_body(h_ref, w1, b1, g1, e1, w2, b2, g2, e2, w3, b3, ws, bs, wh, bh,
              h_o, s_o, hp_o, sq_o):
    y = jnp.maximum(_dot_t(h_ref[...], w1[...]) + b1[...], 0.0)
    y = _bn(y, g1[...], e1[...])
    y = jnp.maximum(_dot_t(y, w2[...]) + b2[...], 0.0)
    y = _bn(y, g2[...], e2[...])
    h = _dot_t(y, w3[...]) + b3[...]
    s = _dot_t(h, ws[...]) + bs[...]
    hp = _dot_t(h, wh[...]) + bh[...]
    h_o[...] = h
    s_o[...] = s
    hp_o[...] = jnp.concatenate(
        [hp, s, jnp.zeros((N, 128 - PROP_D - SPACE_D), jnp.float32)], axis=1)
    sq_o[...] = jnp.sum(s * s, axis=1, keepdims=True)


def _pre(h_in, blk):
    args = (h_in,
            blk["lin1"]["W"], blk["lin1"]["b"].reshape(1, -1),
            blk["bn1"]["w"].reshape(1, -1), blk["bn1"]["b"].reshape(1, -1),
            blk["lin2"]["W"], blk["lin2"]["b"].reshape(1, -1),
            blk["bn2"]["w"].reshape(1, -1), blk["bn2"]["b"].reshape(1, -1),
            blk["lin3"]["W"], blk["lin3"]["b"].reshape(1, -1),
            blk["lin_s"]["W"], blk["lin_s"]["b"].reshape(1, -1),
            blk["lin_h"]["W"], blk["lin_h"]["b"].reshape(1, -1))
    return pl.pallas_call(
        _pre_body,
        out_shape=(jax.ShapeDtypeStruct((N, GRAV_DIM), jnp.float32),
                   jax.ShapeDtypeStruct((N, SPACE_D), jnp.float32),
                   jax.ShapeDtypeStruct((N, 128), jnp.float32),
                   jax.ShapeDtypeStruct((N, 1), jnp.float32)),
    )(*args)


def _knn_body(st_ref, sqt_ref, bt_ref, s_ref, sqr_ref, brow_ref, idx_ref):
    d2 = sqt_ref[...] + sqr_ref[...] - 2.0 * _dot_t(st_ref[...], s_ref[...])
    same = bt_ref[...] == brow_ref[...]
    d2 = jnp.where(same, d2, jnp.inf)
    iota = lax.broadcasted_iota(jnp.int32, (T, N), 1)
    sels = []
    for _ in range(K):
        m = jnp.min(d2, axis=1, keepdims=True)
        sel = jnp.min(jnp.where(d2 == m, iota, N), axis=1, keepdims=True)
        sels.append(sel)
        d2 = jnp.where(iota == sel, jnp.inf, d2)
    idx_ref[...] = jnp.concatenate(sels, axis=1)


def _knn(s, sq_col, sq_row, bcol, brow):
    return pl.pallas_call(
        _knn_body,
        grid=(N // T,),
        in_specs=[
            pl.BlockSpec((T, SPACE_D), lambda i: (i, 0)),
            pl.BlockSpec((T, 1), lambda i: (i, 0)),
            pl.BlockSpec((T, 1), lambda i: (i, 0)),
            pl.BlockSpec((N, SPACE_D), lambda i: (0, 0)),
            pl.BlockSpec((1, N), lambda i: (0, 0)),
            pl.BlockSpec((1, N), lambda i: (0, 0)),
        ],
        out_specs=pl.BlockSpec((T, K), lambda i: (i, 0)),
        out_shape=jax.ShapeDtypeStruct((N, K), jnp.int32),
    )(s, sq_col, bcol, s, sq_row, brow)


def _sc_gather(table, idx_flat):
    info = plsc.get_sparse_core_info()
    nw = info.num_cores * info.num_subcores
    b = idx_flat.shape[0]
    b_w = b // nw
    ch = 128
    nch = b_w // ch
    mesh = plsc.VectorSubcoreMesh(core_axis_name="c", subcore_axis_name="s")

    @functools.partial(
        pl.kernel, mesh=mesh,
        out_type=jax.ShapeDtypeStruct((b, 128), jnp.float32),
        scratch_types=[pltpu.VMEM((ch,), jnp.int32),
                       pltpu.VMEM((ch, 128), jnp.float32),
                       pltpu.SemaphoreType.DMA],
    )
    def gather_k(tab_hbm, idx_hbm, o_hbm, idx_v, rows_v, sem):
        wid = lax.axis_index("s") * info.num_cores + lax.axis_index("c")
        for cb in range(nch):
            base = wid * b_w + cb * ch
            pltpu.sync_copy(idx_hbm.at[pl.ds(base, ch)], idx_v)
            pltpu.async_copy(tab_hbm.at[idx_v], rows_v, sem).wait()
            pltpu.sync_copy(rows_v, o_hbm.at[pl.ds(base, ch)])

    return gather_k(table, idx_flat)


def _post_body(h_ref, s_ref, g_ref, wo, bo, gw, gb, o_ref):
    s = s_ref[...]
    acc = None
    mx = None
    for k in range(K):
        gk = g_ref[k]
        sk = gk[:, PROP_D:PROP_D + SPACE_D]
        hk = gk[:, :PROP_D]
        d2 = jnp.sum((s - sk) ** 2, axis=1, keepdims=True)
        w = jnp.exp(-10.0 * d2)
        m = hk * w
        acc = m if acc is None else acc + m
        mx = m if mx is None else jnp.maximum(mx, m)
    feat = jnp.concatenate([h_ref[...], acc / float(K), mx], axis=1)
    y = _dot_t(feat, wo[...]) + bo[...]
    o_ref[...] = _bn(y, gw[...], gb[...])


def _post(h, s, g, blk):
    return pl.pallas_call(
        _post_body,
        out_shape=jax.ShapeDtypeStruct((N, GRAV_DIM), jnp.float32),
    )(h, s, g,
      blk["lin_out"]["W"], blk["lin_out"]["b"].reshape(1, -1),
      blk["bn_post"]["w"].reshape(1, -1), blk["bn_post"]["b"].reshape(1, -1))


def _head_body(z0_ref, z1_ref, z2_ref, brow_ref, bcol_ref,
               wd0, bd0, gw0, gb0, wd1, bd1, gw1, gb1, wd2, bd2, gw2, gb2,
               wo, bo, o_ref):
    z = jnp.concatenate([z0_ref[...], z1_ref[...], z2_ref[...]], axis=1)
    brow = brow_ref[...]
    e_row = lax.broadcasted_iota(jnp.int32, (NB, N), 0)
    oh_t = (e_row == brow).astype(jnp.float32)
    sums = _dot(oh_t, z, precision=_HI)
    cnt = jnp.sum(oh_t, axis=1, keepdims=True)
    mean = sums / jnp.maximum(cnt, 1.0)
    mn_list, mx_list = [], []
    bcol = bcol_ref[...]
    for e in range(NB):
        me = bcol == e
        mn_list.append(jnp.min(jnp.where(me, z, jnp.inf), axis=0, keepdims=True))
        mx_list.append(jnp.max(jnp.where(me, z, -jnp.inf), axis=0, keepdims=True))
    zz = jnp.concatenate([mean,
                          jnp.concatenate(mn_list, axis=0),
                          jnp.concatenate(mx_list, axis=0),
                          sums], axis=1)
    for wd, bd, gw, gb in ((wd0, bd0, gw0, gb0), (wd1, bd1, gw1, gb1),
                           (wd2, bd2, gw2, gb2)):
        zz = jnp.maximum(_dot_t(zz, wd[...]) + bd[...], 0.0)
        zz = _bn(zz, gw[...], gb[...])
    o_ref[...] = jnp.sum(zz * wo[...], axis=1, keepdims=True) + bo[...]


def _head(z0, z1, z2, brow, bcol, params):
    args = [z0, z1, z2, brow, bcol]
    for d in params["dense"]:
        args += [d["lin"]["W"], d["lin"]["b"].reshape(1, -1),
                 d["bn"]["w"].reshape(1, -1), d["bn"]["b"].reshape(1, -1)]
    args += [params["out"]["W"], params["out"]["b"].reshape(1, -1)]
    return pl.pallas_call(
        _head_body,
        out_shape=jax.ShapeDtypeStruct((NB, 1), jnp.float32),
    )(*args)


def kernel(x, params, batch):
    batch = batch.astype(jnp.int32)
    bcol = batch.reshape(N, 1)
    brow = batch.reshape(1, N)
    h = _exchange(x, bcol, brow)
    outs = []
    for blk in params["blocks"]:
        h, s, tab, sq = _pre(h, blk)
        sq_row = sq.reshape(1, N)
        idx = _knn(s, sq, sq_row, bcol, brow)
        idx_flat = idx.T.reshape(-1)
        g = _sc_gather(tab, idx_flat).reshape(K, N, 128)
        h = _post(h, s, g, blk)
        outs.append(h)
    return _head(outs[0], outs[1], outs[2], brow, bcol, params)
